# speedup vs baseline: 6.0221x; 6.0221x over previous
import functools

import jax
import jax.numpy as jnp
from jax import lax
from jax.experimental import pallas as pl
from jax.experimental.pallas import tpu as pltpu
from jax.experimental.pallas import tpu_sc as plsc

VOCAB = 100000
POS = 1024
DIM = 128
B = 4096
S = 200
N = B * S
NC = 2
NS = 16
NW = NC * NS
RPW = N // NW
SUB = 128
LANES = 16


def _combo_body(pad_ref, pos_ref, out_ref):
    out_ref[...] = pad_ref[...][:, None, :] + pos_ref[...][None, :, :]


def _make_combo(padding_embedding, pos_embedding):
    combo3 = pl.pallas_call(
        _combo_body,
        out_shape=jax.ShapeDtypeStruct((3, S, DIM), jnp.float32),
    )(padding_embedding, pos_embedding[:S])
    return combo3.reshape(3 * S, DIM)


def _sc_body(char_hbm, pad_hbm, emb_hbm, combo_hbm, out_hbm,
             idx_e, idx_c, rows_e, rows_c, sem_e, sem_c):
    wid = lax.axis_index("s") * NC + lax.axis_index("c")
    base = wid * RPW

    def step(g, carry):
        off = base + g * SUB
        pltpu.sync_copy(char_hbm.at[pl.ds(off, SUB)], idx_e)
        pltpu.sync_copy(pad_hbm.at[pl.ds(off, SUB)], idx_c)

        def fix(k, c):
            sl = pl.ds(k * LANES, LANES)
            lane = lax.iota(jnp.int32, LANES)
            s_v = lax.rem(off + k * LANES + lane, S)
            idx_c[sl] = idx_c[sl] * S + s_v
            return c

        lax.fori_loop(0, SUB // LANES, fix, 0)

        ge = pltpu.async_copy(emb_hbm.at[idx_e], rows_e, sem_e)
        gc = pltpu.async_copy(combo_hbm.at[idx_c], rows_c, sem_c)
        ge.wait()
        gc.wait()

        def add_row(r, c):
            for d in range(DIM // LANES):
                sl = pl.ds(d * LANES, LANES)
                rows_e[r, sl] = rows_e[r, sl] + rows_c[r, sl]
            return c

        lax.fori_loop(0, SUB, add_row, 0)

        pltpu.sync_copy(rows_e, out_hbm.at[pl.ds(off, SUB)])
        return carry

    lax.fori_loop(0, RPW // SUB, step, 0)


_sc_lookup = functools.partial(
    pl.kernel,
    mesh=plsc.VectorSubcoreMesh(core_axis_name="c", subcore_axis_name="s"),
    out_type=jax.ShapeDtypeStruct((N, DIM), jnp.float32),
    scratch_types=[
        pltpu.VMEM((SUB,), jnp.int32),
        pltpu.VMEM((SUB,), jnp.int32),
        pltpu.VMEM((SUB, DIM), jnp.float32),
        pltpu.VMEM((SUB, DIM), jnp.float32),
        pltpu.SemaphoreType.DMA,
        pltpu.SemaphoreType.DMA,
    ],
)(_sc_body)


@jax.jit
def kernel(char_ids, pad_ids, embedding, pos_embedding, padding_embedding):
    combo = _make_combo(padding_embedding, pos_embedding)
    char_flat = char_ids.reshape(N).astype(jnp.int32)
    pad_flat = pad_ids.reshape(N).astype(jnp.int32)
    out = _sc_lookup(char_flat, pad_flat, embedding, combo)
    return out.reshape(B, S, DIM)


# speedup vs baseline: 7.5367x; 1.2515x over previous
import functools

import jax
import jax.numpy as jnp
from jax import lax
from jax.experimental import pallas as pl
from jax.experimental.pallas import tpu as pltpu
from jax.experimental.pallas import tpu_sc as plsc

VOCAB = 100000
POS = 1024
DIM = 128
B = 4096
S = 200
N = B * S
NC = 2
NS = 16
NW = NC * NS
RPW = N // NW
SUB = 128
NCH = RPW // SUB
LANES = 16
NSLOT = 3


def _combo_body(pad_ref, pos_ref, out_ref):
    out_ref[...] = pad_ref[...][:, None, :] + pos_ref[...][None, :, :]


def _make_combo(padding_embedding, pos_embedding):
    combo3 = pl.pallas_call(
        _combo_body,
        out_shape=jax.ShapeDtypeStruct((3, S, DIM), jnp.float32),
    )(padding_embedding, pos_embedding[:S])
    return combo3.reshape(3 * S, DIM)


def _sc_body(char_hbm, pad_hbm, emb_hbm, combo_hbm, out_hbm,
             idx_e, idx_c, rows_e, rows_c,
             sem_ie0, sem_ie1, sem_ie2,
             sem_ic0, sem_ic1, sem_ic2,
             sem_ge0, sem_ge1, sem_ge2,
             sem_gc0, sem_gc1, sem_gc2,
             sem_s0, sem_s1, sem_s2):
    sem_ie = (sem_ie0, sem_ie1, sem_ie2)
    sem_ic = (sem_ic0, sem_ic1, sem_ic2)
    sem_ge = (sem_ge0, sem_ge1, sem_ge2)
    sem_gc = (sem_gc0, sem_gc1, sem_gc2)
    sem_s = (sem_s0, sem_s1, sem_s2)

    wid = lax.axis_index("s") * NC + lax.axis_index("c")
    base = wid * RPW
    lane = lax.iota(jnp.int32, LANES)

    def fire_idx(i, s):
        off = base + i * SUB
        pltpu.async_copy(char_hbm.at[pl.ds(off, SUB)], idx_e.at[s], sem_ie[s])
        pltpu.async_copy(pad_hbm.at[pl.ds(off, SUB)], idx_c.at[s], sem_ic[s])

    def wait_idx(i, s):
        off = base + i * SUB
        pltpu.make_async_copy(char_hbm.at[pl.ds(off, SUB)], idx_e.at[s],
                              sem_ie[s]).wait()
        pltpu.make_async_copy(pad_hbm.at[pl.ds(off, SUB)], idx_c.at[s],
                              sem_ic[s]).wait()

    def fix_idx(i, s):
        off = base + i * SUB

        def fix(k, c):
            sl = pl.ds(k * LANES, LANES)
            s_v = lax.rem(off + k * LANES + lane, S)
            idx_c[s, sl] = idx_c[s, sl] * S + s_v
            return c

        lax.fori_loop(0, SUB // LANES, fix, 0)

    def fire_gather(s):
        pltpu.async_copy(emb_hbm.at[idx_e.at[s]], rows_e.at[s], sem_ge[s])
        pltpu.async_copy(combo_hbm.at[idx_c.at[s]], rows_c.at[s], sem_gc[s])

    def wait_gather(s):
        pltpu.make_async_copy(emb_hbm.at[idx_e.at[s]], rows_e.at[s],
                              sem_ge[s]).wait()
        pltpu.make_async_copy(combo_hbm.at[idx_c.at[s]], rows_c.at[s],
                              sem_gc[s]).wait()

    def add_rows(s):
        def add_row(r, c):
            for d in range(DIM // LANES):
                sl = pl.ds(d * LANES, LANES)
                rows_e[s, r, sl] = rows_e[s, r, sl] + rows_c[s, r, sl]
            return c

        lax.fori_loop(0, SUB, add_row, 0)

    def fire_scatter(i, s):
        off = base + i * SUB
        pltpu.async_copy(rows_e.at[s], out_hbm.at[pl.ds(off, SUB)], sem_s[s])

    def wait_scatter(i, s):
        off = base + i * SUB
        pltpu.make_async_copy(rows_e.at[s], out_hbm.at[pl.ds(off, SUB)],
                              sem_s[s]).wait()

    fire_idx(0, 0)
    fire_idx(1, 1)
    wait_idx(0, 0)
    fix_idx(0, 0)
    fire_gather(0)

    def body(t, carry):
        for j in range(NSLOT):
            i = NSLOT * t + j
            s_c = (j - 1) % NSLOT
            s_g = (j + 1) % NSLOT
            s_i = (j + 2) % NSLOT

            @pl.when(i >= 1)
            def _():
                wait_gather(s_c)
                add_rows(s_c)
                fire_scatter(i - 1, s_c)

            @pl.when(i >= 2)
            def _():
                wait_scatter(i - 2, s_g)

            @pl.when(i <= NCH - 3)
            def _():
                fire_idx(i + 2, s_i)

            @pl.when(i <= NCH - 2)
            def _():
                wait_idx(i + 1, s_g)
                fix_idx(i + 1, s_g)
                fire_gather(s_g)

        return carry

    assert (NCH + 1) % NSLOT == 0
    lax.fori_loop(0, (NCH + 1) // NSLOT, body, 0)

    wait_scatter(NCH - 1, (NCH - 1) % NSLOT)


_sc_lookup = functools.partial(
    pl.kernel,
    mesh=plsc.VectorSubcoreMesh(core_axis_name="c", subcore_axis_name="s"),
    out_type=jax.ShapeDtypeStruct((N, DIM), jnp.float32),
    scratch_types=[
        pltpu.VMEM((NSLOT, SUB), jnp.int32),
        pltpu.VMEM((NSLOT, SUB), jnp.int32),
        pltpu.VMEM((NSLOT, SUB, DIM), jnp.float32),
        pltpu.VMEM((NSLOT, SUB, DIM), jnp.float32),
    ] + [pltpu.SemaphoreType.DMA] * 15,
)(_sc_body)


@jax.jit
def kernel(char_ids, pad_ids, embedding, pos_embedding, padding_embedding):
    combo = _make_combo(padding_embedding, pos_embedding)
    char_flat = char_ids.reshape(N).astype(jnp.int32)
    pad_flat = pad_ids.reshape(N).astype(jnp.int32)
    out = _sc_lookup(char_flat, pad_flat, embedding, combo)
    return out.reshape(B, S, DIM)


# speedup vs baseline: 7.5632x; 1.0035x over previous
import functools

import jax
import jax.numpy as jnp
from jax import lax
from jax.experimental import pallas as pl
from jax.experimental.pallas import tpu as pltpu
from jax.experimental.pallas import tpu_sc as plsc

VOCAB = 100000
POS = 1024
DIM = 128
B = 4096
S = 200
N = B * S
NC = 2
NS = 16
NW = NC * NS
RPW = N // NW
SUB = 128
NCH = RPW // SUB
LANES = 16
NSLOT = 3


def _combo_body(pad_ref, pos_ref, out_ref):
    out_ref[...] = pad_ref[...][:, None, :] + pos_ref[...][None, :, :]


def _make_combo(padding_embedding, pos_embedding):
    combo3 = pl.pallas_call(
        _combo_body,
        out_shape=jax.ShapeDtypeStruct((3, S, DIM), jnp.float32),
    )(padding_embedding, pos_embedding[:S])
    return combo3.reshape(3 * S, DIM)


def _sc_body(char_hbm, pad_hbm, emb_hbm, combo_hbm, out_hbm,
             idx_e, idx_c, rows_e,
             sem_ie0, sem_ie1, sem_ie2,
             sem_ic0, sem_ic1, sem_ic2,
             sem_ge0, sem_ge1, sem_ge2,
             sem_gc0, sem_gc1, sem_gc2,
             sem_s0, sem_s1, sem_s2):
    sem_ie = (sem_ie0, sem_ie1, sem_ie2)
    sem_ic = (sem_ic0, sem_ic1, sem_ic2)
    sem_ge = (sem_ge0, sem_ge1, sem_ge2)
    sem_gc = (sem_gc0, sem_gc1, sem_gc2)
    sem_s = (sem_s0, sem_s1, sem_s2)

    wid = lax.axis_index("s") * NC + lax.axis_index("c")
    base = wid * RPW
    lane = lax.iota(jnp.int32, LANES)

    def fire_idx(i, s):
        off = base + i * SUB
        pltpu.async_copy(char_hbm.at[pl.ds(off, SUB)], idx_e.at[s], sem_ie[s])
        pltpu.async_copy(pad_hbm.at[pl.ds(off, SUB)], idx_c.at[s], sem_ic[s])

    def wait_idx(i, s):
        off = base + i * SUB
        pltpu.make_async_copy(char_hbm.at[pl.ds(off, SUB)], idx_e.at[s],
                              sem_ie[s]).wait()
        pltpu.make_async_copy(pad_hbm.at[pl.ds(off, SUB)], idx_c.at[s],
                              sem_ic[s]).wait()

    def fix_idx(i, s):
        off = base + i * SUB

        def fix(k, c):
            sl = pl.ds(k * LANES, LANES)
            s_v = lax.rem(off + k * LANES + lane, S)
            idx_c[s, sl] = idx_c[s, sl] * S + s_v
            return c

        lax.fori_loop(0, SUB // LANES, fix, 0)

    def fire_ge(s):
        pltpu.async_copy(emb_hbm.at[idx_e.at[s]], rows_e.at[s], sem_ge[s])

    def wait_ge(s):
        pltpu.make_async_copy(emb_hbm.at[idx_e.at[s]], rows_e.at[s],
                              sem_ge[s]).wait()

    def fire_gc(s):
        pltpu.async_copy(combo_hbm.at[idx_c.at[s]], rows_e.at[s], sem_gc[s],
                         add=True)

    def wait_gc(s):
        pltpu.make_async_copy(combo_hbm.at[idx_c.at[s]], rows_e.at[s],
                              sem_gc[s]).wait()

    def fire_scatter(i, s):
        off = base + i * SUB
        pltpu.async_copy(rows_e.at[s], out_hbm.at[pl.ds(off, SUB)], sem_s[s])

    def wait_scatter(i, s):
        off = base + i * SUB
        pltpu.make_async_copy(rows_e.at[s], out_hbm.at[pl.ds(off, SUB)],
                              sem_s[s]).wait()

    fire_idx(0, 0)
    fire_idx(1, 1)
    wait_idx(0, 0)
    fix_idx(0, 0)
    fire_ge(0)

    def body(t, carry):
        for j in range(NSLOT):
            i = NSLOT * t + j
            s_0 = j
            s_c = (j - 1) % NSLOT
            s_g = (j + 1) % NSLOT
            s_i = (j + 2) % NSLOT

            @pl.when(i >= 1)
            def _():
                wait_gc(s_c)
                fire_scatter(i - 1, s_c)

            @pl.when(i >= 2)
            def _():
                wait_scatter(i - 2, s_g)

            @pl.when(i <= NCH - 2)
            def _():
                wait_idx(i + 1, s_g)
                fix_idx(i + 1, s_g)
                fire_ge(s_g)

            @pl.when(i <= NCH - 1)
            def _():
                wait_ge(s_0)
                fire_gc(s_0)

            @pl.when(i <= NCH - 3)
            def _():
                fire_idx(i + 2, s_i)

        return carry

    assert (NCH + 1) % NSLOT == 0
    lax.fori_loop(0, (NCH + 1) // NSLOT, body, 0)

    wait_scatter(NCH - 1, (NCH - 1) % NSLOT)


_sc_lookup = functools.partial(
    pl.kernel,
    mesh=plsc.VectorSubcoreMesh(core_axis_name="c", subcore_axis_name="s"),
    out_type=jax.ShapeDtypeStruct((N, DIM), jnp.float32),
    scratch_types=[
        pltpu.VMEM((NSLOT, SUB), jnp.int32),
        pltpu.VMEM((NSLOT, SUB), jnp.int32),
        pltpu.VMEM((NSLOT, SUB, DIM), jnp.float32),
    ] + [pltpu.SemaphoreType.DMA] * 15,
)(_sc_body)


@jax.jit
def kernel(char_ids, pad_ids, embedding, pos_embedding, padding_embedding):
    combo = _make_combo(padding_embedding, pos_embedding)
    char_flat = char_ids.reshape(N).astype(jnp.int32)
    pad_flat = pad_ids.reshape(N).astype(jnp.int32)
    out = _sc_lookup(char_flat, pad_flat, embedding, combo)
    return out.reshape(B, S, DIM)


# speedup vs baseline: 7.5714x; 1.0011x over previous
import functools

import jax
import jax.numpy as jnp
from jax import lax
from jax.experimental import pallas as pl
from jax.experimental.pallas import tpu as pltpu
from jax.experimental.pallas import tpu_sc as plsc

VOCAB = 100000
POS = 1024
DIM = 128
B = 4096
S = 200
N = B * S
NC = 2
NS = 16
NW = NC * NS
RPW = N // NW
SUB = 128
NCH = RPW // SUB
LANES = 16
NSLOT = 5


def _combo_body(pad_ref, pos_ref, out_ref):
    out_ref[...] = pad_ref[...][:, None, :] + pos_ref[...][None, :, :]


def _make_combo(padding_embedding, pos_embedding):
    combo3 = pl.pallas_call(
        _combo_body,
        out_shape=jax.ShapeDtypeStruct((3, S, DIM), jnp.float32),
    )(padding_embedding, pos_embedding[:S])
    return combo3.reshape(3 * S, DIM)


def _sc_body(char_hbm, pad_hbm, emb_hbm, combo_hbm, out_hbm,
             idx_e, idx_c, rows_e, *sems):
    assert len(sems) == 5 * NSLOT
    sem_ie = sems[0 * NSLOT:1 * NSLOT]
    sem_ic = sems[1 * NSLOT:2 * NSLOT]
    sem_ge = sems[2 * NSLOT:3 * NSLOT]
    sem_gc = sems[3 * NSLOT:4 * NSLOT]
    sem_s = sems[4 * NSLOT:5 * NSLOT]

    wid = lax.axis_index("s") * NC + lax.axis_index("c")
    base = wid * RPW
    lane = lax.iota(jnp.int32, LANES)

    def fire_idx(i, s):
        off = base + i * SUB
        pltpu.async_copy(char_hbm.at[pl.ds(off, SUB)], idx_e.at[s], sem_ie[s])
        pltpu.async_copy(pad_hbm.at[pl.ds(off, SUB)], idx_c.at[s], sem_ic[s])

    def wait_idx(i, s):
        off = base + i * SUB
        pltpu.make_async_copy(char_hbm.at[pl.ds(off, SUB)], idx_e.at[s],
                              sem_ie[s]).wait()
        pltpu.make_async_copy(pad_hbm.at[pl.ds(off, SUB)], idx_c.at[s],
                              sem_ic[s]).wait()

    def fix_idx(i, s):
        off = base + i * SUB

        def fix(k, c):
            sl = pl.ds(k * LANES, LANES)
            s_v = lax.rem(off + k * LANES + lane, S)
            idx_c[s, sl] = idx_c[s, sl] * S + s_v
            return c

        lax.fori_loop(0, SUB // LANES, fix, 0)

    def fire_ge(s):
        pltpu.async_copy(emb_hbm.at[idx_e.at[s]], rows_e.at[s], sem_ge[s])

    def wait_ge(s):
        pltpu.make_async_copy(emb_hbm.at[idx_e.at[s]], rows_e.at[s],
                              sem_ge[s]).wait()

    def fire_gc(s):
        pltpu.async_copy(combo_hbm.at[idx_c.at[s]], rows_e.at[s], sem_gc[s],
                         add=True)

    def wait_gc(s):
        pltpu.make_async_copy(combo_hbm.at[idx_c.at[s]], rows_e.at[s],
                              sem_gc[s]).wait()

    def fire_scatter(i, s):
        off = base + i * SUB
        pltpu.async_copy(rows_e.at[s], out_hbm.at[pl.ds(off, SUB)], sem_s[s])

    def wait_scatter(i, s):
        off = base + i * SUB
        pltpu.make_async_copy(rows_e.at[s], out_hbm.at[pl.ds(off, SUB)],
                              sem_s[s]).wait()

    fire_idx(0, 0)
    fire_idx(1, 1)
    wait_idx(0, 0)
    fix_idx(0, 0)
    fire_ge(0)

    def body(t, carry):
        for j in range(NSLOT):
            i = NSLOT * t + j
            s_0 = j
            s_c = (j - 1) % NSLOT
            s_g = (j + 1) % NSLOT
            s_i = (j + 2) % NSLOT

            @pl.when(jnp.logical_and(i >= 1, i <= NCH))
            def _():
                wait_gc(s_c)
                fire_scatter(i - 1, s_c)

            @pl.when(jnp.logical_and(i >= NSLOT - 1, i <= NCH + NSLOT - 2))
            def _():
                wait_scatter(i + 1 - NSLOT, s_g)

            @pl.when(i <= NCH - 2)
            def _():
                wait_idx(i + 1, s_g)
                fix_idx(i + 1, s_g)
                fire_ge(s_g)

            @pl.when(i <= NCH - 1)
            def _():
                wait_ge(s_0)
                fire_gc(s_0)

            @pl.when(i <= NCH - 3)
            def _():
                fire_idx(i + 2, s_i)

        return carry

    n_iter = (NCH + NSLOT + NSLOT - 1) // NSLOT
    lax.fori_loop(0, n_iter, body, 0)


_sc_lookup = functools.partial(
    pl.kernel,
    mesh=plsc.VectorSubcoreMesh(core_axis_name="c", subcore_axis_name="s"),
    out_type=jax.ShapeDtypeStruct((N, DIM), jnp.float32),
    scratch_types=[
        pltpu.VMEM((NSLOT, SUB), jnp.int32),
        pltpu.VMEM((NSLOT, SUB), jnp.int32),
        pltpu.VMEM((NSLOT, SUB, DIM), jnp.float32),
    ] + [pltpu.SemaphoreType.DMA] * (5 * NSLOT),
)(_sc_body)


@jax.jit
def kernel(char_ids, pad_ids, embedding, pos_embedding, padding_embedding):
    combo = _make_combo(padding_embedding, pos_embedding)
    char_flat = char_ids.reshape(N).astype(jnp.int32)
    pad_flat = pad_ids.reshape(N).astype(jnp.int32)
    out = _sc_lookup(char_flat, pad_flat, embedding, combo)
    return out.reshape(B, S, DIM)


# speedup vs baseline: 17.4459x; 2.3042x over previous
import functools

import jax
import jax.numpy as jnp
from jax import lax
from jax.experimental import pallas as pl
from jax.experimental.pallas import tpu as pltpu
from jax.experimental.pallas import tpu_sc as plsc

VOCAB = 100000
POS = 1024
DIM = 128
B = 4096
S = 200
N = B * S
NC = 2
NS = 16
NW = NC * NS
RPW = N // NW
SUB = 128
NCH = RPW // SUB
LANES = 16
NSLOT = 5


def _combo_body(pad_ref, pos_ref, out_ref):
    out_ref[...] = pad_ref[...][:, None, :] + pos_ref[...][None, :, :]


def _make_combo(padding_embedding, pos_embedding):
    combo3 = pl.pallas_call(
        _combo_body,
        out_shape=jax.ShapeDtypeStruct((3, S, DIM), jnp.float32),
    )(padding_embedding, pos_embedding[:S])
    return combo3.reshape(3 * S, DIM)


def _sc_body(char_hbm, pad_hbm, emb_hbm, combo_hbm, out_hbm,
             idx_e, idx_c, rows_e, combo_sh, *sems):
    assert len(sems) == 5 * NSLOT
    sem_ie = sems[0 * NSLOT:1 * NSLOT]
    sem_ic = sems[1 * NSLOT:2 * NSLOT]
    sem_ge = sems[2 * NSLOT:3 * NSLOT]
    sem_gc = sems[3 * NSLOT:4 * NSLOT]
    sem_s = sems[4 * NSLOT:5 * NSLOT]

    wid = lax.axis_index("s") * NC + lax.axis_index("c")
    base = wid * RPW
    lane = lax.iota(jnp.int32, LANES)

    def fire_idx(i, s):
        off = base + i * SUB
        pltpu.async_copy(char_hbm.at[pl.ds(off, SUB)], idx_e.at[s], sem_ie[s])
        pltpu.async_copy(pad_hbm.at[pl.ds(off, SUB)], idx_c.at[s], sem_ic[s])

    def wait_idx(i, s):
        off = base + i * SUB
        pltpu.make_async_copy(char_hbm.at[pl.ds(off, SUB)], idx_e.at[s],
                              sem_ie[s]).wait()
        pltpu.make_async_copy(pad_hbm.at[pl.ds(off, SUB)], idx_c.at[s],
                              sem_ic[s]).wait()

    def fix_idx(i, s):
        off = base + i * SUB

        def fix(k, c):
            sl = pl.ds(k * LANES, LANES)
            s_v = lax.rem(off + k * LANES + lane, S)
            idx_c[s, sl] = idx_c[s, sl] * S + s_v
            return c

        lax.fori_loop(0, SUB // LANES, fix, 0)

    def fire_ge(s):
        pltpu.async_copy(emb_hbm.at[idx_e.at[s]], rows_e.at[s], sem_ge[s])

    def wait_ge(s):
        pltpu.make_async_copy(emb_hbm.at[idx_e.at[s]], rows_e.at[s],
                              sem_ge[s]).wait()

    def fire_gc(s):
        pltpu.async_copy(combo_sh.at[idx_c.at[s]], rows_e.at[s], sem_gc[s],
                         add=True)

    def wait_gc(s):
        pltpu.make_async_copy(combo_sh.at[idx_c.at[s]], rows_e.at[s],
                              sem_gc[s]).wait()

    def fire_scatter(i, s):
        off = base + i * SUB
        pltpu.async_copy(rows_e.at[s], out_hbm.at[pl.ds(off, SUB)], sem_s[s])

    def wait_scatter(i, s):
        off = base + i * SUB
        pltpu.make_async_copy(rows_e.at[s], out_hbm.at[pl.ds(off, SUB)],
                              sem_s[s]).wait()

    @pl.when(lax.axis_index("s") == 0)
    def _():
        pltpu.sync_copy(combo_hbm, combo_sh)

    plsc.subcore_barrier()

    fire_idx(0, 0)
    fire_idx(1, 1)
    wait_idx(0, 0)
    fix_idx(0, 0)
    fire_ge(0)

    def body(t, carry):
        for j in range(NSLOT):
            i = NSLOT * t + j
            s_0 = j
            s_c = (j - 1) % NSLOT
            s_g = (j + 1) % NSLOT
            s_i = (j + 2) % NSLOT

            @pl.when(jnp.logical_and(i >= 1, i <= NCH))
            def _():
                wait_gc(s_c)
                fire_scatter(i - 1, s_c)

            @pl.when(jnp.logical_and(i >= NSLOT - 1, i <= NCH + NSLOT - 2))
            def _():
                wait_scatter(i + 1 - NSLOT, s_g)

            @pl.when(i <= NCH - 2)
            def _():
                wait_idx(i + 1, s_g)
                fix_idx(i + 1, s_g)
                fire_ge(s_g)

            @pl.when(i <= NCH - 1)
            def _():
                wait_ge(s_0)
                fire_gc(s_0)

            @pl.when(i <= NCH - 3)
            def _():
                fire_idx(i + 2, s_i)

        return carry

    n_iter = (NCH + NSLOT + NSLOT - 1) // NSLOT
    lax.fori_loop(0, n_iter, body, 0)


_sc_lookup = functools.partial(
    pl.kernel,
    mesh=plsc.VectorSubcoreMesh(core_axis_name="c", subcore_axis_name="s"),
    out_type=jax.ShapeDtypeStruct((N, DIM), jnp.float32),
    scratch_types=[
        pltpu.VMEM((NSLOT, SUB), jnp.int32),
        pltpu.VMEM((NSLOT, SUB), jnp.int32),
        pltpu.VMEM((NSLOT, SUB, DIM), jnp.float32),
        pltpu.VMEM_SHARED((3 * S, DIM), jnp.float32),
    ] + [pltpu.SemaphoreType.DMA] * (5 * NSLOT),
)(_sc_body)


@jax.jit
def kernel(char_ids, pad_ids, embedding, pos_embedding, padding_embedding):
    combo = _make_combo(padding_embedding, pos_embedding)
    char_flat = char_ids.reshape(N).astype(jnp.int32)
    pad_flat = pad_ids.reshape(N).astype(jnp.int32)
    out = _sc_lookup(char_flat, pad_flat, embedding, combo)
    return out.reshape(B, S, DIM)
